# baseline (speedup 1.0000x reference)
import functools

import numpy as np

import jax
import jax.numpy as jnp
from jax import lax
from jax.experimental import pallas as pl
from jax.experimental.pallas import tpu as pltpu
from jax.experimental.pallas import tpu_sc as plsc

K_KNOTS = 64
_N = 1048576
_NW = 32
_P = _N // _NW
_C = 8192
_NCH = _P // _C

_TB = 2048

_A_ROWS = np.array(
    [[0.0, 1.0, 0.0, 0.0],
     [-0.5, 0.0, 0.5, 0.0],
     [1.0, -2.5, 2.0, -0.5],
     [-0.5, 1.5, -1.5, 0.5]], dtype=np.float32)


def _sc_idx(coord):
    mesh = plsc.VectorSubcoreMesh(core_axis_name="c", subcore_axis_name="s")

    @functools.partial(
        pl.kernel,
        mesh=mesh,
        out_type=jax.ShapeDtypeStruct((4 * _N,), jnp.int32),
        scratch_types=[
            pltpu.VMEM((_C,), jnp.float32),
            pltpu.VMEM((_C,), jnp.float32),
            pltpu.VMEM((4 * _C,), jnp.int32),
            pltpu.VMEM((4 * _C,), jnp.int32),
            pltpu.SemaphoreType.DMA,
            pltpu.SemaphoreType.DMA,
        ],
    )
    def k(coord_hbm, idx_hbm, cv0, cv1, ib0, ib1, in_sem, out_sem):
        wid = lax.axis_index("s") * 2 + lax.axis_index("c")
        base = wid * _P
        cv = (cv0, cv1)
        ib = (ib0, ib1)

        def start_in(ch, b):
            return pltpu.async_copy(
                coord_hbm.at[pl.ds(base + ch * _C, _C)], cv[b], in_sem)

        def start_out(ch, b):
            return pltpu.async_copy(
                ib[b], idx_hbm.at[pl.ds(4 * (base + ch * _C), 4 * _C)],
                out_sem)

        def compute(b):
            coord_v, idx_b = cv[b], ib[b]

            @plsc.parallel_loop(0, _C // 16, unroll=4)
            def body(i):
                c16 = coord_v[pl.ds(i * 16, 16)]
                idx0 = c16.astype(jnp.int32)
                off = (i >> 3) * 512 + (i & 7) * 16
                idx_b[pl.ds(off, 16)] = jnp.maximum(idx0 - 1, 0)
                idx_b[pl.ds(off + 128, 16)] = idx0
                idx_b[pl.ds(off + 256, 16)] = idx0 + 1
                idx_b[pl.ds(off + 384, 16)] = idx0 + 2

        in_d = [None] * _NCH
        out_d = [None] * _NCH
        in_d[0] = start_in(0, 0)
        for ch in range(_NCH):
            b = ch % 2
            in_d[ch].wait()
            if ch + 1 < _NCH:
                in_d[ch + 1] = start_in(ch + 1, 1 - b)
            if ch >= 2:
                out_d[ch - 2].wait()
            compute(b)
            out_d[ch] = start_out(ch, b)
        for ch in range(max(_NCH - 2, 0), _NCH):
            out_d[ch].wait()

    return k(coord)


def _tc_w_body(a_ref, c_ref, w_ref):
    a0 = a_ref[0:8, :]
    a1 = a_ref[8:16, :]
    a2 = a_ref[16:24, :]
    a3 = a_ref[24:32, :]
    def slice_body(r, _):
        c2 = c_ref[pl.ds(2 * r, 2), :]
        idx0 = c2.astype(jnp.int32)
        s2 = c2 - idx0.astype(jnp.float32)
        sr = jnp.repeat(s2, 4, axis=0)
        w_ref[pl.ds(8 * r, 8), :] = a0 + sr * (a1 + sr * (a2 + sr * a3))
        return 0

    lax.fori_loop(0, _TB // 2, slice_body, 0, unroll=8)


def _tc_w(coord2d):
    nrows = _N // 128
    acoef = jnp.asarray(
        np.tile(np.tile(_A_ROWS, (1, 2)).reshape(4, 8, 1), (1, 1, 128))
        .reshape(32, 128))
    return pl.pallas_call(
        _tc_w_body,
        grid=(nrows // _TB,),
        in_specs=[
            pl.BlockSpec((32, 128), lambda i: (0, 0)),
            pl.BlockSpec((_TB, 128), lambda i: (i, 0)),
        ],
        out_specs=pl.BlockSpec((4 * _TB, 128), lambda i: (i, 0)),
        out_shape=jax.ShapeDtypeStruct((4 * nrows, 128), jnp.float32),
    )(acoef, coord2d)


def _unblock(x):
    return x.reshape(_N // 128, 4, 128).transpose(0, 2, 1).reshape(_N, 4)


def kernel(coord, axis):
    idx_flat = _sc_idx(coord)
    w_panels = _tc_w(coord.reshape(_N // 128, 128))
    return _unblock(idx_flat), _unblock(w_panels)


# speedup vs baseline: 1.1679x; 1.1679x over previous
import functools

import numpy as np

import jax
import jax.numpy as jnp
from jax import lax
from jax.experimental import pallas as pl
from jax.experimental.pallas import tpu as pltpu
from jax.experimental.pallas import tpu_sc as plsc

K_KNOTS = 64
_N = 1048576
_NW = 32
_P = _N // _NW
_C = 8192
_NCH = _P // _C

_TB = 2048

_A_ROWS = np.array(
    [[0.0, 1.0, 0.0, 0.0],
     [-0.5, 0.0, 0.5, 0.0],
     [1.0, -2.5, 2.0, -0.5],
     [-0.5, 1.5, -1.5, 0.5]], dtype=np.float32)


def _sc_idx(coord):
    mesh = plsc.VectorSubcoreMesh(core_axis_name="c", subcore_axis_name="s")

    @functools.partial(
        pl.kernel,
        mesh=mesh,
        out_type=jax.ShapeDtypeStruct((4 * _N,), jnp.int32),
        scratch_types=[
            pltpu.VMEM((_C,), jnp.float32),
            pltpu.VMEM((_C,), jnp.float32),
            pltpu.VMEM((4 * _C,), jnp.int32),
            pltpu.VMEM((4 * _C,), jnp.int32),
            pltpu.SemaphoreType.DMA,
            pltpu.SemaphoreType.DMA,
        ],
    )
    def k(coord_hbm, idx_hbm, cv0, cv1, ib0, ib1, in_sem, out_sem):
        wid = lax.axis_index("s") * 2 + lax.axis_index("c")
        base = wid * _P
        cv = (cv0, cv1)
        ib = (ib0, ib1)

        def start_in(ch, b):
            return pltpu.async_copy(
                coord_hbm.at[pl.ds(base + ch * _C, _C)], cv[b], in_sem)

        def start_out(ch, b):
            return pltpu.async_copy(
                ib[b], idx_hbm.at[pl.ds(4 * (base + ch * _C), 4 * _C)],
                out_sem)

        def compute(b):
            coord_v, idx_b = cv[b], ib[b]

            @plsc.parallel_loop(0, _C // 16, unroll=4)
            def body(i):
                c16 = coord_v[pl.ds(i * 16, 16)]
                idx0 = c16.astype(jnp.int32)
                off = (i >> 3) * 512 + (i & 7) * 16
                idx_b[pl.ds(off, 16)] = jnp.maximum(idx0 - 1, 0)
                idx_b[pl.ds(off + 128, 16)] = idx0
                idx_b[pl.ds(off + 256, 16)] = idx0 + 1
                idx_b[pl.ds(off + 384, 16)] = idx0 + 2

        in_d = [None] * _NCH
        out_d = [None] * _NCH
        in_d[0] = start_in(0, 0)
        for ch in range(_NCH):
            b = ch % 2
            in_d[ch].wait()
            if ch + 1 < _NCH:
                in_d[ch + 1] = start_in(ch + 1, 1 - b)
            if ch >= 2:
                out_d[ch - 2].wait()
            compute(b)
            out_d[ch] = start_out(ch, b)
        for ch in range(max(_NCH - 2, 0), _NCH):
            out_d[ch].wait()

    return k(coord)


def _tc_w_body(a_ref, c_ref, w_ref):
    a0 = a_ref[0:8, :]
    a1 = a_ref[8:16, :]
    a2 = a_ref[16:24, :]
    a3 = a_ref[24:32, :]
    half = lax.broadcasted_iota(jnp.int32, (8, 128), 0) < 4

    def slice_body(r, _):
        c2 = c_ref[pl.ds(2 * r, 2), :]
        idx0 = c2.astype(jnp.int32)
        s2 = c2 - idx0.astype(jnp.float32)
        b0 = jnp.broadcast_to(s2[0:1, :], (8, 128))
        b1 = jnp.broadcast_to(s2[1:2, :], (8, 128))
        sr = jnp.where(half, b0, b1)
        w_ref[pl.ds(8 * r, 8), :] = a0 + sr * (a1 + sr * (a2 + sr * a3))
        return 0

    lax.fori_loop(0, _TB // 2, slice_body, 0, unroll=8)


def _tc_w(coord2d):
    nrows = _N // 128
    acoef = jnp.asarray(
        np.tile(np.tile(_A_ROWS, (1, 2)).reshape(4, 8, 1), (1, 1, 128))
        .reshape(32, 128))
    return pl.pallas_call(
        _tc_w_body,
        grid=(nrows // _TB,),
        in_specs=[
            pl.BlockSpec((32, 128), lambda i: (0, 0)),
            pl.BlockSpec((_TB, 128), lambda i: (i, 0)),
        ],
        out_specs=pl.BlockSpec((4 * _TB, 128), lambda i: (i, 0)),
        out_shape=jax.ShapeDtypeStruct((4 * nrows, 128), jnp.float32),
    )(acoef, coord2d)


def _unblock(x):
    return x.reshape(_N // 128, 4, 128).transpose(0, 2, 1).reshape(_N, 4)


def kernel(coord, axis):
    idx_flat = _sc_idx(coord)
    w_panels = _tc_w(coord.reshape(_N // 128, 128))
    return _unblock(idx_flat), _unblock(w_panels)


# speedup vs baseline: 1.1720x; 1.0035x over previous
import functools

import numpy as np

import jax
import jax.numpy as jnp
from jax import lax
from jax.experimental import pallas as pl
from jax.experimental.pallas import tpu as pltpu
from jax.experimental.pallas import tpu_sc as plsc

K_KNOTS = 64
_N = 1048576
_NW = 32
_P = _N // _NW
_C = 8192
_NCH = _P // _C

_TB = 2048


def _sc_w(coord):
    mesh = plsc.VectorSubcoreMesh(core_axis_name="c", subcore_axis_name="s")

    @functools.partial(
        pl.kernel,
        mesh=mesh,
        out_type=jax.ShapeDtypeStruct((4 * _N,), jnp.float32),
        scratch_types=[
            pltpu.VMEM((_C,), jnp.float32),
            pltpu.VMEM((_C,), jnp.float32),
            pltpu.VMEM((4 * _C,), jnp.float32),
            pltpu.VMEM((4 * _C,), jnp.float32),
            pltpu.SemaphoreType.DMA,
            pltpu.SemaphoreType.DMA,
        ],
    )
    def k(coord_hbm, w_hbm, cv0, cv1, wb0, wb1, in_sem, out_sem):
        wid = lax.axis_index("s") * 2 + lax.axis_index("c")
        base = wid * _P
        cv = (cv0, cv1)
        wb = (wb0, wb1)

        def start_in(ch, b):
            return pltpu.async_copy(
                coord_hbm.at[pl.ds(base + ch * _C, _C)], cv[b], in_sem)

        def start_out(ch, b):
            return pltpu.async_copy(
                wb[b], w_hbm.at[pl.ds(4 * (base + ch * _C), 4 * _C)],
                out_sem)

        def compute(b):
            coord_v, w_b = cv[b], wb[b]

            @plsc.parallel_loop(0, _C // 16, unroll=4)
            def body(i):
                c16 = coord_v[pl.ds(i * 16, 16)]
                idx0 = c16.astype(jnp.int32)
                s = c16 - idx0.astype(jnp.float32)
                s2 = s * s
                s3 = s2 * s
                w0 = s2 - 0.5 * (s + s3)
                w3 = 0.5 * (s3 - s2)
                w2 = w0 + (s + s2 - s3)
                w1 = 1.0 - w0 - w2 - w3
                off = (i >> 3) * 512 + (i & 7) * 16
                w_b[pl.ds(off, 16)] = w0
                w_b[pl.ds(off + 128, 16)] = w1
                w_b[pl.ds(off + 256, 16)] = w2
                w_b[pl.ds(off + 384, 16)] = w3

        in_d = [None] * _NCH
        out_d = [None] * _NCH
        in_d[0] = start_in(0, 0)
        for ch in range(_NCH):
            b = ch % 2
            in_d[ch].wait()
            if ch + 1 < _NCH:
                in_d[ch + 1] = start_in(ch + 1, 1 - b)
            if ch >= 2:
                out_d[ch - 2].wait()
            compute(b)
            out_d[ch] = start_out(ch, b)
        for ch in range(max(_NCH - 2, 0), _NCH):
            out_d[ch].wait()

    return k(coord)


def _tc_idx_body(c_ref, i_ref):
    offs = (lax.broadcasted_iota(jnp.int32, (8, 128), 0) % 4) - 1
    half = lax.broadcasted_iota(jnp.int32, (8, 128), 0) < 4

    def slice_body(r, _):
        c2 = c_ref[pl.ds(2 * r, 2), :]
        idx0 = c2.astype(jnp.int32)
        b0 = jnp.broadcast_to(idx0[0:1, :], (8, 128))
        b1 = jnp.broadcast_to(idx0[1:2, :], (8, 128))
        ir = jnp.where(half, b0, b1)
        i_ref[pl.ds(8 * r, 8), :] = jnp.maximum(ir + offs, 0)
        return 0

    lax.fori_loop(0, _TB // 2, slice_body, 0, unroll=8)


def _tc_idx(coord2d):
    nrows = _N // 128
    return pl.pallas_call(
        _tc_idx_body,
        grid=(nrows // _TB,),
        in_specs=[pl.BlockSpec((_TB, 128), lambda i: (i, 0))],
        out_specs=pl.BlockSpec((4 * _TB, 128), lambda i: (i, 0)),
        out_shape=jax.ShapeDtypeStruct((4 * nrows, 128), jnp.int32),
    )(coord2d)


def _unblock(x):
    return x.reshape(_N // 128, 4, 128).transpose(0, 2, 1).reshape(_N, 4)


def kernel(coord, axis):
    w_flat = _sc_w(coord)
    idx_panels = _tc_idx(coord.reshape(_N // 128, 128))
    return _unblock(idx_panels), _unblock(w_flat)
